# baseline (speedup 1.0000x reference)
import jax
import jax.numpy as jnp
from jax import lax
from jax.experimental import pallas as pl
from jax.experimental.pallas import tpu as pltpu
from jax.experimental.pallas import tpu_sc as plsc

EMB = 64
SCALE = 8.0
VOCAB = 1_000_000
NW = 32
NTILE = VOCAB // 128
TPW = (NTILE + NW - 1) // NW
VPAD = NTILE * 128 + 128

NI = 16384
ICHUNK = 256
NJ = 50


def _tp_body(tt_hbm, tail_hbm, tlin_hbm, blk_v, obuf_v, sem):
    c = lax.axis_index("c")
    s = lax.axis_index("s")
    wid = s * 2 + c

    iota = lax.iota(jnp.int32, 16)
    krows = [kg * 16 + iota for kg in range(4)]

    def transpose_block(ncols):
        def body(v, carry):
            cols = jnp.full((16,), v, dtype=jnp.int32)
            for kg in range(4):
                vals = plsc.load_gather(blk_v, [krows[kg], cols])
                obuf_v[v, pl.ds(kg * 16, 16)] = vals * SCALE
            return carry

        lax.fori_loop(0, ncols, body, 0)

    def do_tile(t, carry):
        b = wid + t * NW

        @pl.when(b < NTILE)
        def _():
            pltpu.sync_copy(tt_hbm.at[:, pl.ds(b * 128, 128)], blk_v)
            transpose_block(128)
            pltpu.sync_copy(obuf_v, tlin_hbm.at[pl.ds(b * 128, 128)])

        return carry

    lax.fori_loop(0, TPW, do_tile, 0)

    @pl.when(wid == 0)
    def _():
        pltpu.sync_copy(tail_hbm, blk_v)
        transpose_block(64)
        pltpu.sync_copy(
            obuf_v.at[pl.ds(0, 64)], tlin_hbm.at[pl.ds(NTILE * 128, 64)]
        )


def _gather_body(xl_hbm, tlin_hbm, out_hbm, idx_v, grows_v, brick_v, gsem):
    c = lax.axis_index("c")
    s = lax.axis_index("s")
    wid = s * 2 + c

    iota = lax.iota(jnp.int32, 16)
    irows = [ig * 16 + iota for ig in range(ICHUNK // 16)]

    def one_unit(j, half):
        a0 = 4 * wid + 2 * half
        pltpu.sync_copy(xl_hbm.at[j, pl.ds(a0, 2)], idx_v)
        for h in range(2):
            pltpu.make_async_copy(
                tlin_hbm.at[idx_v.at[h]],
                grows_v.at[pl.ds(h * 128, 128)],
                gsem,
            ).start()
        for h in range(2):
            pltpu.make_async_copy(
                tlin_hbm.at[idx_v.at[0]],
                grows_v.at[pl.ds(h * 128, 128)],
                gsem,
            ).wait()

        def k_body(k, carry):
            cols = jnp.full((16,), k, dtype=jnp.int32)
            for ig in range(ICHUNK // 16):
                vals = plsc.load_gather(grows_v, [irows[ig], cols])
                brick_v[k, pl.ds(ig * 16, 16)] = vals
            return carry

        lax.fori_loop(0, EMB, k_body, 0)

        pltpu.sync_copy(
            brick_v,
            out_hbm.at[j, :, pl.ds(wid * 512 + half * ICHUNK, ICHUNK)],
        )

    def j_loop(j, carry):
        one_unit(j, 0)
        one_unit(j, 1)
        return carry

    lax.fori_loop(0, NJ, j_loop, 0)


def kernel(x, table):
    mesh = plsc.VectorSubcoreMesh(core_axis_name="c", subcore_axis_name="s")

    tt = table.T
    tail = jnp.pad(tt[:, NTILE * 128:], ((0, 0), (0, 64)))

    tlin = pl.kernel(
        _tp_body,
        out_type=jax.ShapeDtypeStruct((VPAD, 128), jnp.float32),
        mesh=mesh,
        compiler_params=pltpu.CompilerParams(use_tc_tiling_on_sc=True, needs_layout_passes=False),
        scratch_types=[
            pltpu.VMEM((64, 128), jnp.float32),
            pltpu.VMEM((128, 128), jnp.float32),
            pltpu.SemaphoreType.DMA,
        ],
    )(tt, tail)

    xl = x.T.astype(jnp.int32).reshape(NJ, NI // 128, 128)

    ot = pl.kernel(
        _gather_body,
        out_type=jax.ShapeDtypeStruct((NJ, EMB, NI), jnp.float32),
        mesh=mesh,
        compiler_params=pltpu.CompilerParams(use_tc_tiling_on_sc=True, needs_layout_passes=False),
        scratch_types=[
            pltpu.VMEM((2, 128), jnp.int32),
            pltpu.VMEM((ICHUNK, 128), jnp.float32),
            pltpu.VMEM((EMB, ICHUNK), jnp.float32),
            pltpu.SemaphoreType.DMA,
        ],
    )(xl, tlin)

    return jnp.transpose(ot, (2, 0, 1))


# speedup vs baseline: 2.4734x; 2.4734x over previous
import jax
import jax.numpy as jnp
from jax import lax
from jax.experimental import pallas as pl
from jax.experimental.pallas import tpu as pltpu
from jax.experimental.pallas import tpu_sc as plsc

EMB = 64
SCALE = 8.0
VOCAB = 1_000_000
NW = 32
NTILE = VOCAB // 128
TPW = (NTILE + NW - 1) // NW
VPAD = NTILE * 128 + 128

NI = 16384
ICHUNK = 256
NJ = 50


def _tp_body(tt_hbm, tail_hbm, tlin_hbm, blk_v, obuf_v, rsem, osem):
    c = lax.axis_index("c")
    s = lax.axis_index("s")
    wid = s * 2 + c

    iota = lax.iota(jnp.int32, 16)
    krows = [kg * 16 + iota for kg in range(4)]

    def bnum(t):
        return wid + t * NW

    def fire_read(t, p):
        @pl.when(bnum(t) < NTILE)
        def _():
            pltpu.make_async_copy(
                tt_hbm.at[:, pl.ds(bnum(t) * 128, 128)],
                blk_v.at[p],
                rsem.at[p],
            ).start()

    def wait_read(t, p):
        @pl.when(bnum(t) < NTILE)
        def _():
            pltpu.make_async_copy(
                tt_hbm.at[:, pl.ds(0, 128)], blk_v.at[p], rsem.at[p]
            ).wait()

    def out_desc(t, p):
        return pltpu.make_async_copy(
            obuf_v.at[p], tlin_hbm.at[pl.ds(bnum(t) * 128, 128)], osem.at[p]
        )

    def transpose_block(p, ncols):
        @plsc.parallel_loop(0, ncols, unroll=4)
        def _(v):
            cols = jnp.full((16,), v, dtype=jnp.int32)
            for kg in range(4):
                vals = plsc.load_gather(blk_v.at[p], [krows[kg], cols])
                obuf_v[p, v, pl.ds(kg * 16, 16)] = vals * SCALE

    def step(t, p):
        wait_read(t, p)

        @pl.when(bnum(t) < NTILE)
        def _():
            @pl.when(t >= 2)
            def _():
                out_desc(t, p).wait()

            transpose_block(p, 128)
            out_desc(t, p).start()

        fire_read(t + 2, p)

    fire_read(0, 0)
    fire_read(1, 1)

    def blk2(i, carry):
        step(2 * i, 0)
        step(2 * i + 1, 1)
        return carry

    lax.fori_loop(0, TPW // 2, blk2, 0)
    step(TPW - 1, 0)

    @pl.when(bnum(TPW - 1) < NTILE)
    def _():
        out_desc(0, 0).wait()

    @pl.when(bnum(TPW - 2) < NTILE)
    def _():
        out_desc(0, 1).wait()

    @pl.when(wid == 0)
    def _():
        pltpu.sync_copy(tail_hbm, blk_v.at[0])
        transpose_block(0, 64)
        pltpu.sync_copy(
            obuf_v.at[0, pl.ds(0, 64)], tlin_hbm.at[pl.ds(NTILE * 128, 64)]
        )


def _gather_body(xl_hbm, tlin_hbm, out_hbm, idx_v, grows_v, brick_v, gsem, osem):
    c = lax.axis_index("c")
    s = lax.axis_index("s")
    wid = s * 2 + c

    iota = lax.iota(jnp.int32, 16)
    irows = [ig * 16 + iota for ig in range(ICHUNK // 16)]

    pltpu.sync_copy(xl_hbm.at[:, pl.ds(4 * wid, 4)], idx_v)

    def fire_gathers(j, half):
        for h in range(2):
            pltpu.make_async_copy(
                tlin_hbm.at[idx_v.at[j, 2 * half + h]],
                grows_v.at[half, pl.ds(h * 128, 128)],
                gsem.at[half],
            ).start()

    def wait_gathers(half):
        for h in range(2):
            pltpu.make_async_copy(
                tlin_hbm.at[idx_v.at[0, 0]],
                grows_v.at[half, pl.ds(h * 128, 128)],
                gsem.at[half],
            ).wait()

    def brick_desc(j, half):
        return pltpu.make_async_copy(
            brick_v.at[half],
            out_hbm.at[j, :, pl.ds(wid * 512 + half * ICHUNK, ICHUNK)],
            osem.at[half],
        )

    def transpose_unit(half):
        @plsc.parallel_loop(0, EMB, unroll=4)
        def _(k):
            cols = jnp.full((16,), k, dtype=jnp.int32)
            for ig in range(ICHUNK // 16):
                vals = plsc.load_gather(grows_v.at[half], [irows[ig], cols])
                brick_v[half, k, pl.ds(ig * 16, 16)] = vals

    def unit(j, half, jn, halfn, first, last):
        wait_gathers(half)
        if not last:
            fire_gathers(jn, halfn)
        if not first:
            brick_desc(0, half).wait()
        transpose_unit(half)
        brick_desc(j, half).start()

    fire_gathers(0, 0)

    def j_loop(j, carry):
        @pl.when(j == 0)
        def _():
            fire_gathers(0, 1)
            wait_gathers(0)
            transpose_unit(0)
            brick_desc(0, 0).start()
            fire_gathers(1, 0)
            wait_gathers(1)
            transpose_unit(1)
            brick_desc(0, 1).start()

        @pl.when(j > 0)
        def _():
            unit(j, 0, j, 1, False, False)
            unit(j, 1, j + 1, 0, False, False)

        return carry

    lax.fori_loop(0, NJ - 1, j_loop, 0)
    unit(NJ - 1, 0, NJ - 1, 1, False, False)
    unit(NJ - 1, 1, 0, 0, False, True)

    brick_desc(0, 0).wait()
    brick_desc(0, 1).wait()


def kernel(x, table):
    mesh = plsc.VectorSubcoreMesh(core_axis_name="c", subcore_axis_name="s")

    tt = table.T
    tail = jnp.pad(tt[:, NTILE * 128:], ((0, 0), (0, 64)))

    tlin = pl.kernel(
        _tp_body,
        out_type=jax.ShapeDtypeStruct((VPAD, 128), jnp.float32),
        mesh=mesh,
        compiler_params=pltpu.CompilerParams(
            use_tc_tiling_on_sc=True, needs_layout_passes=False
        ),
        scratch_types=[
            pltpu.VMEM((2, 64, 128), jnp.float32),
            pltpu.VMEM((2, 128, 128), jnp.float32),
            pltpu.SemaphoreType.DMA((2,)),
            pltpu.SemaphoreType.DMA((2,)),
        ],
    )(tt, tail)

    xl = x.T.astype(jnp.int32).reshape(NJ, NI // 128, 128)

    ot = pl.kernel(
        _gather_body,
        out_type=jax.ShapeDtypeStruct((NJ, EMB, NI), jnp.float32),
        mesh=mesh,
        compiler_params=pltpu.CompilerParams(
            use_tc_tiling_on_sc=True, needs_layout_passes=False
        ),
        scratch_types=[
            pltpu.VMEM((NJ, 4, 128), jnp.int32),
            pltpu.VMEM((2, ICHUNK, 128), jnp.float32),
            pltpu.VMEM((2, EMB, ICHUNK), jnp.float32),
            pltpu.SemaphoreType.DMA((2,)),
            pltpu.SemaphoreType.DMA((2,)),
        ],
    )(xl, tlin)

    return jnp.transpose(ot, (2, 0, 1))
